# baseline (speedup 1.0000x reference)
import functools

import numpy as np
import jax
import jax.numpy as jnp
from jax import lax
from jax.experimental import pallas as pl
from jax.experimental.pallas import tpu as pltpu
from jax.experimental.pallas import tpu_sc as plsc

_R = 64
_D = 32
_K = 512
_B = 1024
_NI = 16
_N = _B * _NI
_N8 = _N * 8
_COMMIT = 0.25

_t_np = np.linspace(0.05, 1.2, _NI + 1, dtype=np.float32)
_tmid_np = 0.5 * (_t_np[:-1] + _t_np[1:])
_dt_np = (_t_np[1:] - _t_np[:-1]).astype(np.float32)
_offsets_np = np.array(
    [[i, j, k] for i in (0, 1) for j in (0, 1) for k in (0, 1)], dtype=np.int32
)

_NW = 32

_PW = _N8 // _NW
_CH = 1024
_G = 128


def _sc_gather(table, idx):
    mesh = plsc.VectorSubcoreMesh(core_axis_name="c", subcore_axis_name="s")

    @functools.partial(
        pl.kernel,
        mesh=mesh,
        out_type=jax.ShapeDtypeStruct((_N8, _D), jnp.float32),
        scratch_types=[
            pltpu.VMEM((_PW,), jnp.int32),
            pltpu.VMEM((_CH, _D), jnp.float32),
            pltpu.SemaphoreType.DMA,
        ],
        compiler_params=pltpu.CompilerParams(use_tc_tiling_on_sc=False),
    )
    def k(table_hbm, idx_hbm, out_hbm, idx_v, rows_v, sem):
        wid = lax.axis_index("s") * 2 + lax.axis_index("c")
        base = wid * _PW
        j = wid // (_N // _PW)
        off = (j // 4) * ((_R + 1) * (_R + 1)) + ((j // 2) % 2) * (_R + 1) + (j % 2)
        n0 = (wid % (_N // _PW)) * _PW
        pltpu.sync_copy(idx_hbm.at[pl.ds(n0, _PW)], idx_v)

        def addoff(i, carry):
            s = pl.ds(i * 16, 16)
            idx_v[s] = idx_v[s] + off
            return carry

        lax.fori_loop(0, _PW // 16, addoff, 0)
        for s in range(_PW // _CH):
            cps = [
                pltpu.async_copy(
                    table_hbm.at[idx_v.at[pl.ds(s * _CH + c * _G, _G)]],
                    rows_v.at[pl.ds(c * _G, _G)],
                    sem,
                )
                for c in range(_CH // _G)
            ]
            for cp in cps:
                cp.wait()
            pltpu.sync_copy(rows_v, out_hbm.at[pl.ds(base + s * _CH, _CH)])

    return k(table, idx)


_TN = 1024
_NT8 = _N8 // _TN


def _vq_body(z_ref, vr_ref, cm2_ref, cn_ref,
             idx_ref, loss_ref, ms_ref):
    t = pl.program_id(0)
    z = z_ref[...]
    score = lax.dot_general(
        cm2_ref[...], z, (((1,), (1,)), ((), ())),
        preferred_element_type=jnp.float32,
    ) + cn_ref[...]
    mn = jnp.min(score, axis=0).reshape(1, _TN)
    idx = jnp.argmin(score, axis=0)
    idx_ref[...] = idx.reshape(1, 1, _TN)

    vr = vr_ref[...]
    zz = jnp.sum(z * z, axis=1, keepdims=True)
    part = lax.dot_general(
        vr, zz, (((1,), (0,)), ((), ())),
        preferred_element_type=jnp.float32,
    ) + jnp.sum(vr * mn).reshape(1, 1)

    @pl.when(t == 0)
    def _init():
        loss_ref[...] = jnp.zeros_like(loss_ref)
        ms_ref[...] = jnp.zeros_like(ms_ref)

    loss_ref[...] += part
    ms_ref[...] += jnp.sum(vr).reshape(1, 1)


def _run_vq(zflat, vrow8, cm2, cn):
    return pl.pallas_call(
        _vq_body,
        grid=(_NT8,),
        in_specs=[
            pl.BlockSpec((_TN, _D), lambda t: (t, 0)),
            pl.BlockSpec((1, _TN), lambda t: (0, t)),
            pl.BlockSpec((_K, _D), lambda t: (0, 0)),
            pl.BlockSpec((_K, 1), lambda t: (0, 0)),
        ],
        out_specs=[
            pl.BlockSpec((1, 1, _TN), lambda t: (t, 0, 0)),
            pl.BlockSpec((1, 1), lambda t: (0, 0)),
            pl.BlockSpec((1, 1), lambda t: (0, 0)),
        ],
        out_shape=[
            jax.ShapeDtypeStruct((_NT8, 1, _TN), jnp.int32),
            jax.ShapeDtypeStruct((1, 1), jnp.float32),
            jax.ShapeDtypeStruct((1, 1), jnp.float32),
        ],
    )(zflat, vrow8, cm2, cn)


_PW2 = _N // _NW
_L = 16


def _sc_interp(idx8, wv8, vmask, c4t):
    mesh = plsc.VectorSubcoreMesh(core_axis_name="c", subcore_axis_name="s")

    @functools.partial(
        pl.kernel,
        mesh=mesh,
        out_type=[
            jax.ShapeDtypeStruct((4, _N), jnp.float32),
            jax.ShapeDtypeStruct((_NW, _K), jnp.float32),
        ],
        scratch_types=[
            pltpu.VMEM((4, _K), jnp.float32),
            pltpu.VMEM((8, _PW2), jnp.int32),
            pltpu.VMEM((8, _PW2), jnp.float32),
            pltpu.VMEM((_PW2,), jnp.float32),
            pltpu.VMEM((4, _PW2), jnp.float32),
            pltpu.VMEM((_K,), jnp.float32),
        ],
        compiler_params=pltpu.CompilerParams(
            use_tc_tiling_on_sc=False, needs_layout_passes=False),
    )
    def k(idx_hbm, wv_hbm, v_hbm, c4_hbm, interp_hbm, cnt_hbm,
          c4_v, idx_v, wv_v, v_v, out_v, cnt_v):
        wid = lax.axis_index("s") * 2 + lax.axis_index("c")
        base = wid * _PW2
        pltpu.sync_copy(c4_hbm, c4_v)
        pltpu.sync_copy(idx_hbm.at[:, pl.ds(base, _PW2)], idx_v)
        pltpu.sync_copy(wv_hbm.at[:, pl.ds(base, _PW2)], wv_v)
        pltpu.sync_copy(v_hbm.at[pl.ds(base, _PW2)], v_v)
        for i in range(_K // _L):
            cnt_v[pl.ds(i * _L, _L)] = jnp.zeros((_L,), jnp.float32)

        def body(g, carry):
            s = pl.ds(g * _L, _L)
            vg = v_v[s]
            accs = [jnp.zeros((_L,), jnp.float32) for _ in range(4)]
            for j in range(8):
                ixg = idx_v[j, s]
                wg = wv_v[j, s]
                for c in range(4):
                    accs[c] = accs[c] + wg * plsc.load_gather(c4_v, [
                        jnp.full((_L,), c, jnp.int32), ixg])
                plsc.addupdate_scatter(cnt_v, [ixg], vg)
            for c in range(4):
                out_v[c, s] = accs[c]
            return carry

        lax.fori_loop(0, _PW2 // _L, body, 0)
        pltpu.sync_copy(out_v, interp_hbm.at[:, pl.ds(base, _PW2)])
        pltpu.sync_copy(cnt_v, cnt_hbm.at[wid])

    return k(idx8, wv8, vmask, c4t)


def _sigm(x):
    return 1.0 / (1.0 + jnp.exp(-x))


def _fin_body(t4_ref, cparts_ref, loss_ref, ms_ref,
              vql_ref, perp_ref, r0_ref, r1_ref, r2_ref):
    ms8 = ms_ref[...]
    vql_ref[...] = (1.0 + _COMMIT) * loss_ref[...] / (ms8 * _D + 1e-9)
    counts = jnp.sum(cparts_ref[...], axis=0, keepdims=True)
    avg = counts / (ms8 + 1e-9)
    perp_ref[...] = jnp.exp(-jnp.sum(avg * jnp.log(avg + 1e-10))).reshape(1, 1)

    trans = jnp.ones((1, _B), jnp.float32)
    a0 = jnp.zeros((1, _B), jnp.float32)
    a1 = jnp.zeros((1, _B), jnp.float32)
    a2 = jnp.zeros((1, _B), jnp.float32)
    for i in range(_NI):
        blk = t4_ref[i]
        sig = jnp.maximum(blk[0:1, :], 0.0)
        alpha = 1.0 - jnp.exp(-sig * float(_dt_np[i]))
        w = alpha * trans
        a0 = a0 + w * _sigm(blk[1:2, :])
        a1 = a1 + w * _sigm(blk[2:3, :])
        a2 = a2 + w * _sigm(blk[3:4, :])
        trans = trans * (1.0 - alpha + 1e-10)
    r0_ref[...] = a0
    r1_ref[...] = a1
    r2_ref[...] = a2


def _run_fin(t4, cparts, loss, ms):
    return pl.pallas_call(
        _fin_body,
        out_shape=[
            jax.ShapeDtypeStruct((1, 1), jnp.float32),
            jax.ShapeDtypeStruct((1, 1), jnp.float32),
            jax.ShapeDtypeStruct((1, _B), jnp.float32),
            jax.ShapeDtypeStruct((1, _B), jnp.float32),
            jax.ShapeDtypeStruct((1, _B), jnp.float32),
        ],
    )(t4, cparts, loss, ms)


def _prep(rays_o, rays_d):
    tm = jnp.asarray(_tmid_np)
    pts = rays_o[:, None, :] + tm[None, :, None] * rays_d[:, None, :]
    valid = jnp.all((pts >= 0.0) & (pts < 1.0), axis=-1)
    flat_pts = pts.reshape(-1, 3)
    clipped = jnp.clip(flat_pts, 0.0, 1.0 - 1e-6)
    scaled = clipped * _R
    idx0f = jnp.clip(jnp.floor(scaled), 0.0, _R - 1)
    frac = scaled - idx0f
    idx0 = idx0f.astype(jnp.int32)
    offs = jnp.asarray(_offsets_np)
    nid0 = (idx0[:, 0] * (_R + 1) + idx0[:, 1]) * (_R + 1) + idx0[:, 2]
    w = jnp.where(offs[:, None, :] == 1, frac[None, :, :], 1.0 - frac[None, :, :])
    iw8 = jnp.prod(w, axis=-1)
    return nid0, iw8, valid


def kernel(rays_o, rays_d, data_weight, codebook):
    nid0, iw8, valid = _prep(rays_o, rays_d)

    vf = valid.reshape(-1).astype(jnp.float32)
    vrow8 = jnp.tile(vf, 8).reshape(1, _N8)
    wv8 = iw8 * vf[None, :]

    zflat = _sc_gather(data_weight, nid0)

    cm2 = -2.0 * codebook
    cn = jnp.sum(codebook * codebook, axis=1).reshape(_K, 1)
    c4t = codebook[:, 0:4].T

    idx_out, loss, ms = _run_vq(zflat, vrow8, cm2, cn)
    idx8 = idx_out.reshape(8, _N)

    interp_t, cparts = _sc_interp(idx8, wv8, vf, c4t)

    t4 = interp_t.reshape(4, _B, _NI).transpose(2, 0, 1)
    vql, perp, r0, r1, r2 = _run_fin(t4, cparts, loss, ms)
    rgb = jnp.concatenate([r0, r1, r2], axis=0).T
    return (vql[0, 0], perp[0, 0], rgb)


# speedup vs baseline: 1.0421x; 1.0421x over previous
import functools

import numpy as np
import jax
import jax.numpy as jnp
from jax import lax
from jax.experimental import pallas as pl
from jax.experimental.pallas import tpu as pltpu
from jax.experimental.pallas import tpu_sc as plsc

_R = 64
_D = 32
_K = 512
_B = 1024
_NI = 16
_N = _B * _NI
_N8 = _N * 8
_COMMIT = 0.25

_t_np = np.linspace(0.05, 1.2, _NI + 1, dtype=np.float32)
_tmid_np = 0.5 * (_t_np[:-1] + _t_np[1:])
_dt_np = (_t_np[1:] - _t_np[:-1]).astype(np.float32)
_offsets_np = np.array(
    [[i, j, k] for i in (0, 1) for j in (0, 1) for k in (0, 1)], dtype=np.int32
)

_NW = 32

_PW = _N8 // _NW
_CH = 1024
_G = 128


def _sc_gather(table, idx):
    mesh = plsc.VectorSubcoreMesh(core_axis_name="c", subcore_axis_name="s")

    @functools.partial(
        pl.kernel,
        mesh=mesh,
        out_type=jax.ShapeDtypeStruct((_N8, _D), jnp.float32),
        scratch_types=[
            pltpu.VMEM((_PW,), jnp.int32),
            pltpu.VMEM((_CH, _D), jnp.float32),
            pltpu.SemaphoreType.DMA,
        ],
        compiler_params=pltpu.CompilerParams(use_tc_tiling_on_sc=False),
    )
    def k(table_hbm, idx_hbm, out_hbm, idx_v, rows_v, sem):
        wid = lax.axis_index("s") * 2 + lax.axis_index("c")
        base = wid * _PW
        j = wid // (_N // _PW)
        off = (j // 4) * ((_R + 1) * (_R + 1)) + ((j // 2) % 2) * (_R + 1) + (j % 2)
        n0 = (wid % (_N // _PW)) * _PW
        pltpu.sync_copy(idx_hbm.at[pl.ds(n0, _PW)], idx_v)

        def addoff(i, carry):
            s = pl.ds(i * 16, 16)
            idx_v[s] = idx_v[s] + off
            return carry

        lax.fori_loop(0, _PW // 16, addoff, 0)
        for s in range(_PW // _CH):
            cps = [
                pltpu.async_copy(
                    table_hbm.at[idx_v.at[pl.ds(s * _CH + c * _G, _G)]],
                    rows_v.at[pl.ds(c * _G, _G)],
                    sem,
                )
                for c in range(_CH // _G)
            ]
            for cp in cps:
                cp.wait()
            pltpu.sync_copy(rows_v, out_hbm.at[pl.ds(base + s * _CH, _CH)])

    return k(table, idx)


_TN = 1024
_NT8 = _N8 // _TN


def _vq_body(z_ref, cm2_ref, cn_ref, idx_ref, dmin_ref):
    z = z_ref[...]
    score = lax.dot_general(
        cm2_ref[...], z, (((1,), (1,)), ((), ())),
        preferred_element_type=jnp.float32,
    ) + cn_ref[...]
    mn = jnp.min(score, axis=0).reshape(1, _TN)
    idx = jnp.argmin(score, axis=0)
    idx_ref[...] = idx.reshape(1, 1, _TN)
    zz = lax.dot_general(
        jnp.ones((1, _D), jnp.float32), z * z, (((1,), (1,)), ((), ())),
        preferred_element_type=jnp.float32,
    )
    dmin_ref[...] = (zz + mn).reshape(1, 1, _TN)


def _run_vq(zflat, cm2, cn):
    return pl.pallas_call(
        _vq_body,
        grid=(_NT8,),
        in_specs=[
            pl.BlockSpec((_TN, _D), lambda t: (t, 0)),
            pl.BlockSpec((_K, _D), lambda t: (0, 0)),
            pl.BlockSpec((_K, 1), lambda t: (0, 0)),
        ],
        out_specs=[
            pl.BlockSpec((1, 1, _TN), lambda t: (t, 0, 0)),
            pl.BlockSpec((1, 1, _TN), lambda t: (t, 0, 0)),
        ],
        out_shape=[
            jax.ShapeDtypeStruct((_NT8, 1, _TN), jnp.int32),
            jax.ShapeDtypeStruct((_NT8, 1, _TN), jnp.float32),
        ],
    )(zflat, cm2, cn)


_PW2 = _N // _NW
_L = 16


def _sc_interp(idx3, dmin3, wv8, vmask, c4t):
    mesh = plsc.VectorSubcoreMesh(core_axis_name="c", subcore_axis_name="s")

    @functools.partial(
        pl.kernel,
        mesh=mesh,
        out_type=[
            jax.ShapeDtypeStruct((4, _N), jnp.float32),
            jax.ShapeDtypeStruct((_NW, _K), jnp.float32),
            jax.ShapeDtypeStruct((_NW, _L), jnp.float32),
        ],
        scratch_types=[
            pltpu.VMEM((4, _K), jnp.float32),
            pltpu.VMEM((8, _PW2), jnp.int32),
            pltpu.VMEM((8, _PW2), jnp.float32),
            pltpu.VMEM((8, _PW2), jnp.float32),
            pltpu.VMEM((_PW2,), jnp.float32),
            pltpu.VMEM((4, _PW2), jnp.float32),
            pltpu.VMEM((_K,), jnp.float32),
            pltpu.VMEM((_L,), jnp.float32),
        ],
        compiler_params=pltpu.CompilerParams(
            use_tc_tiling_on_sc=False, needs_layout_passes=False),
    )
    def k(idx_hbm, dm_hbm, wv_hbm, v_hbm, c4_hbm,
          interp_hbm, cnt_hbm, lp_hbm,
          c4_v, idx_v, dm_v, wv_v, v_v, out_v, cnt_v, lp_v):
        wid = lax.axis_index("s") * 2 + lax.axis_index("c")
        base = wid * _PW2
        pltpu.sync_copy(c4_hbm, c4_v)
        for j in range(8):
            r0 = j * _N + base
            tj = r0 // _TN
            o = r0 % _TN
            pltpu.sync_copy(idx_hbm.at[tj, 0, pl.ds(o, _PW2)], idx_v.at[j])
            pltpu.sync_copy(dm_hbm.at[tj, 0, pl.ds(o, _PW2)], dm_v.at[j])
        pltpu.sync_copy(wv_hbm.at[:, pl.ds(base, _PW2)], wv_v)
        pltpu.sync_copy(v_hbm.at[pl.ds(base, _PW2)], v_v)
        for i in range(_K // _L):
            cnt_v[pl.ds(i * _L, _L)] = jnp.zeros((_L,), jnp.float32)

        def body(g, lacc):
            s = pl.ds(g * _L, _L)
            vg = v_v[s]
            accs = [jnp.zeros((_L,), jnp.float32) for _ in range(4)]
            for j in range(8):
                ixg = idx_v[j, s]
                wg = wv_v[j, s]
                lacc = lacc + vg * dm_v[j, s]
                for c in range(4):
                    accs[c] = accs[c] + wg * plsc.load_gather(c4_v, [
                        jnp.full((_L,), c, jnp.int32), ixg])
                plsc.addupdate_scatter(cnt_v, [ixg], vg)
            for c in range(4):
                out_v[c, s] = accs[c]
            return lacc

        lacc = lax.fori_loop(0, _PW2 // _L, body, jnp.zeros((_L,), jnp.float32))
        lp_v[...] = lacc
        pltpu.sync_copy(out_v, interp_hbm.at[:, pl.ds(base, _PW2)])
        pltpu.sync_copy(cnt_v, cnt_hbm.at[wid])
        pltpu.sync_copy(lp_v, lp_hbm.at[wid])

    return k(idx3, dmin3, wv8, vmask, c4t)


def _sigm(x):
    return 1.0 / (1.0 + jnp.exp(-x))


def _fin_body(t4_ref, cparts_ref, lparts_ref,
              vql_ref, perp_ref, r0_ref, r1_ref, r2_ref):
    counts = jnp.sum(cparts_ref[...], axis=0, keepdims=True)
    ms8 = jnp.sum(counts).reshape(1, 1)
    loss = jnp.sum(lparts_ref[...]).reshape(1, 1)
    vql_ref[...] = (1.0 + _COMMIT) * loss / (ms8 * _D + 1e-9)
    avg = counts / (ms8 + 1e-9)
    perp_ref[...] = jnp.exp(-jnp.sum(avg * jnp.log(avg + 1e-10))).reshape(1, 1)

    trans = jnp.ones((1, _B), jnp.float32)
    a0 = jnp.zeros((1, _B), jnp.float32)
    a1 = jnp.zeros((1, _B), jnp.float32)
    a2 = jnp.zeros((1, _B), jnp.float32)
    for i in range(_NI):
        blk = t4_ref[i]
        sig = jnp.maximum(blk[0:1, :], 0.0)
        alpha = 1.0 - jnp.exp(-sig * float(_dt_np[i]))
        w = alpha * trans
        a0 = a0 + w * _sigm(blk[1:2, :])
        a1 = a1 + w * _sigm(blk[2:3, :])
        a2 = a2 + w * _sigm(blk[3:4, :])
        trans = trans * (1.0 - alpha + 1e-10)
    r0_ref[...] = a0
    r1_ref[...] = a1
    r2_ref[...] = a2


def _run_fin(t4, cparts, lparts):
    return pl.pallas_call(
        _fin_body,
        out_shape=[
            jax.ShapeDtypeStruct((1, 1), jnp.float32),
            jax.ShapeDtypeStruct((1, 1), jnp.float32),
            jax.ShapeDtypeStruct((1, _B), jnp.float32),
            jax.ShapeDtypeStruct((1, _B), jnp.float32),
            jax.ShapeDtypeStruct((1, _B), jnp.float32),
        ],
    )(t4, cparts, lparts)


def _prep(rays_o, rays_d):
    tm = jnp.asarray(_tmid_np)
    pts = rays_o[:, None, :] + tm[None, :, None] * rays_d[:, None, :]
    valid = jnp.all((pts >= 0.0) & (pts < 1.0), axis=-1)
    flat_pts = pts.reshape(-1, 3)
    clipped = jnp.clip(flat_pts, 0.0, 1.0 - 1e-6)
    scaled = clipped * _R
    idx0f = jnp.clip(jnp.floor(scaled), 0.0, _R - 1)
    frac = scaled - idx0f
    idx0 = idx0f.astype(jnp.int32)
    offs = jnp.asarray(_offsets_np)
    nid0 = (idx0[:, 0] * (_R + 1) + idx0[:, 1]) * (_R + 1) + idx0[:, 2]
    w = jnp.where(offs[:, None, :] == 1, frac[None, :, :], 1.0 - frac[None, :, :])
    iw8 = jnp.prod(w, axis=-1)
    return nid0, iw8, valid


def kernel(rays_o, rays_d, data_weight, codebook):
    nid0, iw8, valid = _prep(rays_o, rays_d)

    vf = valid.reshape(-1).astype(jnp.float32)
    wv8 = iw8 * vf[None, :]

    zflat = _sc_gather(data_weight, nid0)

    cm2 = -2.0 * codebook
    cn = jnp.sum(codebook * codebook, axis=1).reshape(_K, 1)
    c4t = codebook[:, 0:4].T

    idx_out, dmin_out = _run_vq(zflat, cm2, cn)

    interp_t, cparts, lparts = _sc_interp(idx_out, dmin_out, wv8, vf, c4t)

    t4 = interp_t.reshape(4, _B, _NI).transpose(2, 0, 1)
    vql, perp, r0, r1, r2 = _run_fin(t4, cparts, lparts)
    rgb = jnp.concatenate([r0, r1, r2], axis=0).T
    return (vql[0, 0], perp[0, 0], rgb)


# speedup vs baseline: 1.2173x; 1.1680x over previous
import functools

import numpy as np
import jax
import jax.numpy as jnp
from jax import lax
from jax.experimental import pallas as pl
from jax.experimental.pallas import tpu as pltpu
from jax.experimental.pallas import tpu_sc as plsc

_R = 64
_D = 32
_K = 512
_B = 1024
_NI = 16
_N = _B * _NI
_N8 = _N * 8
_COMMIT = 0.25

_t_np = np.linspace(0.05, 1.2, _NI + 1, dtype=np.float32)
_tmid_np = 0.5 * (_t_np[:-1] + _t_np[1:])
_dt_np = (_t_np[1:] - _t_np[:-1]).astype(np.float32)
_offsets_np = np.array(
    [[i, j, k] for i in (0, 1) for j in (0, 1) for k in (0, 1)], dtype=np.int32
)

_NW = 32

_PW = _N8 // _NW
_CH = 1024
_G = 128


def _sc_gather(table, idx):
    mesh = plsc.VectorSubcoreMesh(core_axis_name="c", subcore_axis_name="s")

    @functools.partial(
        pl.kernel,
        mesh=mesh,
        out_type=jax.ShapeDtypeStruct((_N8, _D), jnp.float32),
        scratch_types=[
            pltpu.VMEM((_PW,), jnp.int32),
            pltpu.VMEM((_CH, _D), jnp.float32),
            pltpu.SemaphoreType.DMA,
        ],
        compiler_params=pltpu.CompilerParams(use_tc_tiling_on_sc=False),
    )
    def k(table_hbm, idx_hbm, out_hbm, idx_v, rows_v, sem):
        wid = lax.axis_index("s") * 2 + lax.axis_index("c")
        base = wid * _PW
        j = wid // (_N // _PW)
        off = (j // 4) * ((_R + 1) * (_R + 1)) + ((j // 2) % 2) * (_R + 1) + (j % 2)
        n0 = (wid % (_N // _PW)) * _PW
        pltpu.sync_copy(idx_hbm.at[pl.ds(n0, _PW)], idx_v)

        def addoff(i, carry):
            s = pl.ds(i * 16, 16)
            idx_v[s] = idx_v[s] + off
            return carry

        lax.fori_loop(0, _PW // 16, addoff, 0)
        for s in range(_PW // _CH):
            cps = [
                pltpu.async_copy(
                    table_hbm.at[idx_v.at[pl.ds(s * _CH + c * _G, _G)]],
                    rows_v.at[pl.ds(c * _G, _G)],
                    sem,
                )
                for c in range(_CH // _G)
            ]
            for cp in cps:
                cp.wait()
            pltpu.sync_copy(rows_v, out_hbm.at[pl.ds(base + s * _CH, _CH)])

    return k(table, idx)


_TN = 1024
_NT8 = _N8 // _TN
_TP = _TN // 4


def _vq_body(z_ref, cm2_ref, cn_ref, idx_ref, dmin_ref):
    z4 = z_ref[...]
    idx_rows = []
    dm_rows = []
    for p in range(4):
        zp = z4[:, 32 * p:32 * (p + 1)]
        score = lax.dot_general(
            cm2_ref[...], zp, (((1,), (1,)), ((), ())),
            preferred_element_type=jnp.float32,
        ) + cn_ref[...]
        mn = jnp.min(score, axis=0).reshape(1, _TP)
        idxp = jnp.argmin(score, axis=0)
        zz = lax.dot_general(
            jnp.ones((1, _D), jnp.float32), zp * zp, (((1,), (1,)), ((), ())),
            preferred_element_type=jnp.float32,
        )
        idx_rows.append(idxp.reshape(1, _TP))
        dm_rows.append(zz + mn)
    idx_ref[...] = jnp.concatenate(idx_rows, axis=0).reshape(1, 4, _TP)
    dmin_ref[...] = jnp.concatenate(dm_rows, axis=0).reshape(1, 4, _TP)


def _run_vq(z4, cm2, cn):
    return pl.pallas_call(
        _vq_body,
        grid=(_NT8,),
        in_specs=[
            pl.BlockSpec((_TP, 128), lambda t: (t, 0)),
            pl.BlockSpec((_K, _D), lambda t: (0, 0)),
            pl.BlockSpec((_K, 1), lambda t: (0, 0)),
        ],
        out_specs=[
            pl.BlockSpec((1, 4, _TP), lambda t: (t, 0, 0)),
            pl.BlockSpec((1, 4, _TP), lambda t: (t, 0, 0)),
        ],
        out_shape=[
            jax.ShapeDtypeStruct((_NT8, 4, _TP), jnp.int32),
            jax.ShapeDtypeStruct((_NT8, 4, _TP), jnp.float32),
        ],
    )(z4, cm2, cn)


_PW2 = _N // _NW
_L = 16


def _sc_interp(idx3, dmin3, wv8, vmask, c4t):
    mesh = plsc.VectorSubcoreMesh(core_axis_name="c", subcore_axis_name="s")

    @functools.partial(
        pl.kernel,
        mesh=mesh,
        out_type=[
            jax.ShapeDtypeStruct((4, _N), jnp.float32),
            jax.ShapeDtypeStruct((_NW, _K), jnp.float32),
            jax.ShapeDtypeStruct((_NW, _L), jnp.float32),
        ],
        scratch_types=[
            pltpu.VMEM((4, _K), jnp.float32),
            pltpu.VMEM((8, 4, _PW2 // 4), jnp.int32),
            pltpu.VMEM((8, 4, _PW2 // 4), jnp.float32),
            pltpu.VMEM((8, _PW2), jnp.float32),
            pltpu.VMEM((_PW2,), jnp.float32),
            pltpu.VMEM((4, _PW2), jnp.float32),
            pltpu.VMEM((_K,), jnp.float32),
            pltpu.VMEM((_L,), jnp.float32),
        ],
        compiler_params=pltpu.CompilerParams(
            use_tc_tiling_on_sc=False, needs_layout_passes=False),
    )
    def k(idx_hbm, dm_hbm, wv_hbm, v_hbm, c4_hbm,
          interp_hbm, cnt_hbm, lp_hbm,
          c4_v, idx_v, dm_v, wv_v, v_v, out_v, cnt_v, lp_v):
        wid = lax.axis_index("s") * 2 + lax.axis_index("c")
        base = wid * _PW2
        pltpu.sync_copy(c4_hbm, c4_v)
        for j in range(8):
            r0 = j * _N + base
            tj = r0 // _TN
            cw = pl.multiple_of((r0 % _TN) // 4, _PW2 // 4)
            pltpu.sync_copy(idx_hbm.at[tj, :, pl.ds(cw, _PW2 // 4)], idx_v.at[j])
            pltpu.sync_copy(dm_hbm.at[tj, :, pl.ds(cw, _PW2 // 4)], dm_v.at[j])
        pltpu.sync_copy(wv_hbm.at[:, pl.ds(base, _PW2)], wv_v)
        pltpu.sync_copy(v_hbm.at[pl.ds(base, _PW2)], v_v)
        for i in range(_K // _L):
            cnt_v[pl.ds(i * _L, _L)] = jnp.zeros((_L,), jnp.float32)

        iot = lax.iota(jnp.int32, _L)
        pv = iot % 4
        cv0 = iot // 4

        def body(g, lacc):
            s = pl.ds(g * _L, _L)
            vg = v_v[s]
            cv = cv0 + 4 * g
            accs = [jnp.zeros((_L,), jnp.float32) for _ in range(4)]
            for j in range(8):
                jv = jnp.full((_L,), j, jnp.int32)
                ixg = plsc.load_gather(idx_v, [jv, pv, cv])
                dmg = plsc.load_gather(dm_v, [jv, pv, cv])
                wg = wv_v[j, s]
                lacc = lacc + vg * dmg
                for c in range(4):
                    accs[c] = accs[c] + wg * plsc.load_gather(c4_v, [
                        jnp.full((_L,), c, jnp.int32), ixg])
                plsc.addupdate_scatter(cnt_v, [ixg], vg)
            for c in range(4):
                out_v[c, s] = accs[c]
            return lacc

        lacc = lax.fori_loop(0, _PW2 // _L, body, jnp.zeros((_L,), jnp.float32))
        lp_v[...] = lacc
        pltpu.sync_copy(out_v, interp_hbm.at[:, pl.ds(base, _PW2)])
        pltpu.sync_copy(cnt_v, cnt_hbm.at[wid])
        pltpu.sync_copy(lp_v, lp_hbm.at[wid])

    return k(idx3, dmin3, wv8, vmask, c4t)


def _sigm(x):
    return 1.0 / (1.0 + jnp.exp(-x))


def _fin_body(t4_ref, cparts_ref, lparts_ref,
              vql_ref, perp_ref, r0_ref, r1_ref, r2_ref):
    counts = jnp.sum(cparts_ref[...], axis=0, keepdims=True)
    ms8 = jnp.sum(counts).reshape(1, 1)
    loss = jnp.sum(lparts_ref[...]).reshape(1, 1)
    vql_ref[...] = (1.0 + _COMMIT) * loss / (ms8 * _D + 1e-9)
    avg = counts / (ms8 + 1e-9)
    perp_ref[...] = jnp.exp(-jnp.sum(avg * jnp.log(avg + 1e-10))).reshape(1, 1)

    trans = jnp.ones((1, _B), jnp.float32)
    a0 = jnp.zeros((1, _B), jnp.float32)
    a1 = jnp.zeros((1, _B), jnp.float32)
    a2 = jnp.zeros((1, _B), jnp.float32)
    for i in range(_NI):
        blk = t4_ref[i]
        sig = jnp.maximum(blk[0:1, :], 0.0)
        alpha = 1.0 - jnp.exp(-sig * float(_dt_np[i]))
        w = alpha * trans
        a0 = a0 + w * _sigm(blk[1:2, :])
        a1 = a1 + w * _sigm(blk[2:3, :])
        a2 = a2 + w * _sigm(blk[3:4, :])
        trans = trans * (1.0 - alpha + 1e-10)
    r0_ref[...] = a0
    r1_ref[...] = a1
    r2_ref[...] = a2


def _run_fin(t4, cparts, lparts):
    return pl.pallas_call(
        _fin_body,
        out_shape=[
            jax.ShapeDtypeStruct((1, 1), jnp.float32),
            jax.ShapeDtypeStruct((1, 1), jnp.float32),
            jax.ShapeDtypeStruct((1, _B), jnp.float32),
            jax.ShapeDtypeStruct((1, _B), jnp.float32),
            jax.ShapeDtypeStruct((1, _B), jnp.float32),
        ],
    )(t4, cparts, lparts)


def _prep(rays_o, rays_d):
    tm = jnp.asarray(_tmid_np)
    pts = rays_o[:, None, :] + tm[None, :, None] * rays_d[:, None, :]
    valid = jnp.all((pts >= 0.0) & (pts < 1.0), axis=-1)
    flat_pts = pts.reshape(-1, 3)
    clipped = jnp.clip(flat_pts, 0.0, 1.0 - 1e-6)
    scaled = clipped * _R
    idx0f = jnp.clip(jnp.floor(scaled), 0.0, _R - 1)
    frac = scaled - idx0f
    idx0 = idx0f.astype(jnp.int32)
    offs = jnp.asarray(_offsets_np)
    nid0 = (idx0[:, 0] * (_R + 1) + idx0[:, 1]) * (_R + 1) + idx0[:, 2]
    w = jnp.where(offs[:, None, :] == 1, frac[None, :, :], 1.0 - frac[None, :, :])
    iw8 = jnp.prod(w, axis=-1)
    return nid0, iw8, valid


def kernel(rays_o, rays_d, data_weight, codebook):
    nid0, iw8, valid = _prep(rays_o, rays_d)

    vf = valid.reshape(-1).astype(jnp.float32)
    wv8 = iw8 * vf[None, :]

    zflat = _sc_gather(data_weight, nid0)

    cm2 = -2.0 * codebook
    cn = jnp.sum(codebook * codebook, axis=1).reshape(_K, 1)
    c4t = codebook[:, 0:4].T

    z4 = zflat.reshape(_N8 // 4, 128)
    idx_out, dmin_out = _run_vq(z4, cm2, cn)

    interp_t, cparts, lparts = _sc_interp(idx_out, dmin_out, wv8, vf, c4t)

    t4 = interp_t.reshape(4, _B, _NI).transpose(2, 0, 1)
    vql, perp, r0, r1, r2 = _run_fin(t4, cparts, lparts)
    rgb = jnp.concatenate([r0, r1, r2], axis=0).T
    return (vql[0, 0], perp[0, 0], rgb)
